# baseline (speedup 1.0000x reference)
import functools

import jax
import jax.numpy as jnp
from jax.experimental import pallas as pl
from jax.experimental.pallas import tpu as pltpu

_K = 1024
_C = 256
_N = 8192
_BN = 1024
_GRID = _N // _BN


def _vq_body(xf_ref, cb_ref, out_ref, idx_ref, loss_ref, usage_ref, counts_ref,
             cbsq_ref):
    i = pl.program_id(0)
    xb = xf_ref[...]
    cb = cb_ref[...]
    x_sq = jnp.sum(xb ** 2, axis=-1, keepdims=True)

    @pl.when(i == 0)
    def _first():
        cb_sq0 = jnp.sum(cb ** 2, axis=-1)
        cbsq_ref[...] = cb_sq0[None, :]

    xc = jax.lax.dot_general(xb, cb, (((1,), (1,)), ((), ())))
    d2 = x_sq + cbsq_ref[...] - 2.0 * xc
    m = jnp.min(d2, axis=1, keepdims=True)
    col = jax.lax.broadcasted_iota(jnp.int32, d2.shape, 1)
    idx = jnp.min(jnp.where(d2 == m, col, _K), axis=1)
    idx_ref[...] = idx.reshape(idx_ref.shape)
    onehot = (col == idx[:, None]).astype(jnp.float32)
    q = jax.lax.dot_general(onehot, cb, (((1,), (0,)), ((), ())))
    out_ref[...] = xb + (q - xb)

    @pl.when(i == 0)
    def _init():
        loss_ref[...] = jnp.zeros_like(loss_ref)
        counts_ref[...] = jnp.zeros_like(counts_ref)

    loss_ref[...] += jnp.sum(m).reshape(1, 1)
    ones_row = jnp.ones((8, _BN), jnp.float32)
    counts_ref[...] += jax.lax.dot_general(
        ones_row, onehot, (((1,), (0,)), ((), ())))[:1]

    @pl.when(i == _GRID - 1)
    def _finish():
        zero_cnt = jnp.sum((counts_ref[...] == 0.0).astype(jnp.float32))
        usage_ref[...] = (zero_cnt / _K).reshape(1, 1)


def _vq_call(xf, codebook):
    return pl.pallas_call(
        _vq_body,
        grid=(_GRID,),
        in_specs=[
            pl.BlockSpec((_BN, _C), lambda i: (i, 0)),
            pl.BlockSpec((_K, _C), lambda i: (0, 0)),
        ],
        out_specs=[
            pl.BlockSpec((_BN, _C), lambda i: (i, 0)),
            pl.BlockSpec((1, 1, _BN), lambda i: (i, 0, 0)),
            pl.BlockSpec((1, 1), lambda i: (0, 0)),
            pl.BlockSpec((1, 1), lambda i: (0, 0)),
            pl.BlockSpec((1, _K), lambda i: (0, 0)),
        ],
        out_shape=[
            jax.ShapeDtypeStruct((_N, _C), jnp.float32),
            jax.ShapeDtypeStruct((_GRID, 1, _BN), jnp.int32),
            jax.ShapeDtypeStruct((1, 1), jnp.float32),
            jax.ShapeDtypeStruct((1, 1), jnp.float32),
            jax.ShapeDtypeStruct((1, _K), jnp.float32),
        ],
        scratch_shapes=[pltpu.VMEM((1, _K), jnp.float32)],
    )(xf, codebook)


def kernel(x, codebook):
    x = x.astype(jnp.float32)
    B, C, H, W = x.shape
    xf = jnp.transpose(x.reshape(B, C, H * W), (0, 2, 1)).reshape(_N, C)
    q_st, idx3, loss_sum, usage, _counts = _vq_call(xf, codebook)
    embed_index = idx3.reshape(B, H, W)
    quantize = jnp.transpose(q_st.reshape(B, H * W, C), (0, 2, 1)).reshape(B, C, H, W)
    loss = (loss_sum / float(_N * _C)).reshape(1)
    code_usage = usage.reshape(())
    return (quantize, embed_index, loss, code_usage)


# speedup vs baseline: 1.0164x; 1.0164x over previous
import functools

import jax
import jax.numpy as jnp
from jax.experimental import pallas as pl
from jax.experimental.pallas import tpu as pltpu

_K = 1024
_C = 256
_N = 8192
_BN = 1024
_GRID = _N // _BN


def _vq_body(xf_ref, cb_ref, out_ref, idx_ref, loss_ref, usage_ref, counts_ref,
             cbsq_ref):
    i = pl.program_id(0)
    xb = xf_ref[...]
    cb = cb_ref[...]
    x_sq = jnp.sum(xb ** 2, axis=-1, keepdims=True)

    @pl.when(i == 0)
    def _first():
        cb_sq0 = jnp.sum(cb ** 2, axis=-1)
        cbsq_ref[...] = cb_sq0[None, :]

    xc = jax.lax.dot_general(xb, cb, (((1,), (1,)), ((), ())))
    d2 = x_sq + cbsq_ref[...] - 2.0 * xc
    m = jnp.min(d2, axis=1, keepdims=True)
    col = jax.lax.broadcasted_iota(jnp.int32, d2.shape, 1)
    idx = jnp.min(jnp.where(d2 == m, col, _K), axis=1)
    idx_ref[...] = idx.reshape(idx_ref.shape)
    onehot = (col == idx[:, None]).astype(jnp.float32)
    q = jax.lax.dot_general(onehot, cb, (((1,), (0,)), ((), ())))
    out_ref[...] = xb + (q - xb)

    @pl.when(i == 0)
    def _init():
        loss_ref[...] = jnp.zeros_like(loss_ref)
        counts_ref[...] = jnp.zeros_like(counts_ref)

    loss_ref[...] += jnp.sum(m).reshape(1, 1)
    counts_ref[...] += jnp.sum(onehot, axis=0, keepdims=True)

    @pl.when(i == _GRID - 1)
    def _finish():
        zero_cnt = jnp.sum((counts_ref[...] == 0.0).astype(jnp.float32))
        usage_ref[...] = (zero_cnt / _K).reshape(1, 1)


def _vq_call(xf, codebook):
    return pl.pallas_call(
        _vq_body,
        grid=(_GRID,),
        in_specs=[
            pl.BlockSpec((_BN, _C), lambda i: (i, 0)),
            pl.BlockSpec((_K, _C), lambda i: (0, 0)),
        ],
        out_specs=[
            pl.BlockSpec((_BN, _C), lambda i: (i, 0)),
            pl.BlockSpec((1, 1, _BN), lambda i: (i, 0, 0)),
            pl.BlockSpec((1, 1), lambda i: (0, 0)),
            pl.BlockSpec((1, 1), lambda i: (0, 0)),
            pl.BlockSpec((1, _K), lambda i: (0, 0)),
        ],
        out_shape=[
            jax.ShapeDtypeStruct((_N, _C), jnp.float32),
            jax.ShapeDtypeStruct((_GRID, 1, _BN), jnp.int32),
            jax.ShapeDtypeStruct((1, 1), jnp.float32),
            jax.ShapeDtypeStruct((1, 1), jnp.float32),
            jax.ShapeDtypeStruct((1, _K), jnp.float32),
        ],
        scratch_shapes=[pltpu.VMEM((1, _K), jnp.float32)],
    )(xf, codebook)


def kernel(x, codebook):
    x = x.astype(jnp.float32)
    B, C, H, W = x.shape
    xf = jnp.transpose(x.reshape(B, C, H * W), (0, 2, 1)).reshape(_N, C)
    q_st, idx3, loss_sum, usage, _counts = _vq_call(xf, codebook)
    embed_index = idx3.reshape(B, H, W)
    quantize = jnp.transpose(q_st.reshape(B, H * W, C), (0, 2, 1)).reshape(B, C, H, W)
    loss = (loss_sum / float(_N * _C)).reshape(1)
    code_usage = usage.reshape(())
    return (quantize, embed_index, loss, code_usage)


# speedup vs baseline: 1.1125x; 1.0946x over previous
import functools

import jax
import jax.numpy as jnp
from jax.experimental import pallas as pl
from jax.experimental.pallas import tpu as pltpu

_K = 1024
_C = 256
_N = 8192
_BN = 2048
_GRID = _N // _BN


def _vq_body(xf_ref, cb_ref, out_ref, idx_ref, loss_ref, usage_ref, counts_ref):
    i = pl.program_id(0)
    xb = xf_ref[...]
    cb = cb_ref[...]
    x_sq = jnp.sum(xb ** 2, axis=-1, keepdims=True)
    cb_sq = jnp.sum(cb ** 2, axis=-1)
    xc = jax.lax.dot_general(xb, cb, (((1,), (1,)), ((), ())))
    d2 = x_sq + cb_sq[None, :] - 2.0 * xc
    m = jnp.min(d2, axis=1, keepdims=True)
    col = jax.lax.broadcasted_iota(jnp.int32, d2.shape, 1)
    idx = jnp.min(jnp.where(d2 == m, col, _K), axis=1)
    idx_ref[...] = idx.reshape(idx_ref.shape)
    onehot = (col == idx[:, None]).astype(jnp.float32)
    q = jax.lax.dot_general(onehot, cb, (((1,), (0,)), ((), ())))
    out_ref[...] = xb + (q - xb)

    @pl.when(i == 0)
    def _init():
        loss_ref[...] = jnp.zeros_like(loss_ref)
        counts_ref[...] = jnp.zeros_like(counts_ref)

    loss_ref[...] += jnp.sum(m).reshape(1, 1)
    counts_ref[...] += jnp.sum(onehot, axis=0, keepdims=True)

    @pl.when(i == _GRID - 1)
    def _finish():
        zero_cnt = jnp.sum((counts_ref[...] == 0.0).astype(jnp.float32))
        usage_ref[...] = (zero_cnt / _K).reshape(1, 1)


def _vq_call(xf, codebook):
    return pl.pallas_call(
        _vq_body,
        grid=(_GRID,),
        in_specs=[
            pl.BlockSpec((_BN, _C), lambda i: (i, 0)),
            pl.BlockSpec((_K, _C), lambda i: (0, 0)),
        ],
        out_specs=[
            pl.BlockSpec((_BN, _C), lambda i: (i, 0)),
            pl.BlockSpec((1, 1, _BN), lambda i: (i, 0, 0)),
            pl.BlockSpec((1, 1), lambda i: (0, 0)),
            pl.BlockSpec((1, 1), lambda i: (0, 0)),
            pl.BlockSpec((1, _K), lambda i: (0, 0)),
        ],
        out_shape=[
            jax.ShapeDtypeStruct((_N, _C), jnp.float32),
            jax.ShapeDtypeStruct((_GRID, 1, _BN), jnp.int32),
            jax.ShapeDtypeStruct((1, 1), jnp.float32),
            jax.ShapeDtypeStruct((1, 1), jnp.float32),
            jax.ShapeDtypeStruct((1, _K), jnp.float32),
        ],
    )(xf, codebook)


def kernel(x, codebook):
    x = x.astype(jnp.float32)
    B, C, H, W = x.shape
    xf = jnp.transpose(x.reshape(B, C, H * W), (0, 2, 1)).reshape(_N, C)
    q_st, idx3, loss_sum, usage, _counts = _vq_call(xf, codebook)
    embed_index = idx3.reshape(B, H, W)
    quantize = jnp.transpose(q_st.reshape(B, H * W, C), (0, 2, 1)).reshape(B, C, H, W)
    loss = (loss_sum / float(_N * _C)).reshape(1)
    code_usage = usage.reshape(())
    return (quantize, embed_index, loss, code_usage)
